# Initial kernel scaffold; baseline (speedup 1.0000x reference)
import jax
import jax.numpy as jnp
from jax.experimental import pallas as pl


def kernel(tokens, table):
    raise NotImplementedError("write your pallas kernel here")


# speedup vs baseline: 1.4175x; 1.4175x over previous
import math

import jax
import jax.numpy as jnp
from jax import lax
from jax.experimental import pallas as pl
from jax.experimental.pallas import tpu as pltpu
from jax.experimental.pallas import tpu_sc as plsc

_D = 32
_B = 4096 * 200
_NC, _NS = 2, 16
_NW = _NC * _NS
_BPW = _B // _NW
_C = 1600
_NCH = _BPW // _C
_SCALE = math.sqrt(float(_D))


def _body(tok_hbm, table_hbm, out_hbm, idx_v, rows_v, sem):
    wid = lax.axis_index("s") * _NC + lax.axis_index("c")
    base = wid * _BPW

    @pl.loop(0, _NCH)
    def _chunk(c):
        off = base + c * _C
        pltpu.sync_copy(tok_hbm.at[pl.ds(off, _C)], idx_v)
        pltpu.async_copy(table_hbm.at[idx_v], rows_v, sem).wait()

        @plsc.parallel_loop(0, _C, 1, unroll=8)
        def _scale(i):
            rows_v[i, pl.ds(0, 16)] = rows_v[i, pl.ds(0, 16)] * _SCALE
            rows_v[i, pl.ds(16, 16)] = rows_v[i, pl.ds(16, 16)] * _SCALE

        pltpu.sync_copy(rows_v, out_hbm.at[pl.ds(off, _C)])


_mesh = plsc.VectorSubcoreMesh(
    core_axis_name="c", subcore_axis_name="s", num_cores=_NC, num_subcores=_NS
)

_gather = pl.kernel(
    _body,
    out_type=jax.ShapeDtypeStruct((_B, _D), jnp.float32),
    mesh=_mesh,
    scratch_types=[
        pltpu.VMEM((_C,), jnp.int32),
        pltpu.VMEM((_C, _D), jnp.float32),
        pltpu.SemaphoreType.DMA,
    ],
    compiler_params=pltpu.CompilerParams(use_tc_tiling_on_sc=False),
)


@jax.jit
def kernel(tokens, table):
    flat = tokens.reshape(-1).astype(jnp.int32)
    out = _gather(flat, table)
    return out.reshape(tokens.shape + (_D,))


# speedup vs baseline: 1.4726x; 1.0389x over previous
import math

import jax
import jax.numpy as jnp
from jax import lax
from jax.experimental import pallas as pl
from jax.experimental.pallas import tpu as pltpu
from jax.experimental.pallas import tpu_sc as plsc

_D = 32
_B = 4096 * 200
_NC, _NS = 2, 16
_NW = _NC * _NS
_BPW = _B // _NW
_C = 1600
_NCH = _BPW // _C
_SCALE = math.sqrt(float(_D))


def _scale_buf(rows_v):
    @plsc.parallel_loop(0, _C, 1, unroll=8)
    def _s(i):
        rows_v[i, pl.ds(0, 16)] = rows_v[i, pl.ds(0, 16)] * _SCALE
        rows_v[i, pl.ds(16, 16)] = rows_v[i, pl.ds(16, 16)] * _SCALE


def _body(tok_hbm, table_hbm, out_hbm,
          idx0, idx1, rows0, rows1, gsem0, gsem1, osem0, osem1):
    idx = (idx0, idx1)
    rows = (rows0, rows1)
    gsem = (gsem0, gsem1)
    osem = (osem0, osem1)
    wid = lax.axis_index("s") * _NC + lax.axis_index("c")
    base = wid * _BPW

    gd = [None, None]
    od = [None, None]
    for c in range(_NCH):
        b = c & 1
        if od[b] is not None:
            od[b].wait()
        pltpu.sync_copy(tok_hbm.at[pl.ds(base + c * _C, _C)], idx[b])
        gd[b] = pltpu.async_copy(table_hbm.at[idx[b]], rows[b], gsem[b])
        if c > 0:
            pb = (c - 1) & 1
            gd[pb].wait()
            _scale_buf(rows[pb])
            od[pb] = pltpu.async_copy(
                rows[pb], out_hbm.at[pl.ds(base + (c - 1) * _C, _C)], osem[pb])
    pb = (_NCH - 1) & 1
    gd[pb].wait()
    _scale_buf(rows[pb])
    od[pb] = pltpu.async_copy(
        rows[pb], out_hbm.at[pl.ds(base + (_NCH - 1) * _C, _C)], osem[pb])
    od[0].wait()
    od[1].wait()


_mesh = plsc.VectorSubcoreMesh(
    core_axis_name="c", subcore_axis_name="s", num_cores=_NC, num_subcores=_NS
)

_gather = pl.kernel(
    _body,
    out_type=jax.ShapeDtypeStruct((_B, _D), jnp.float32),
    mesh=_mesh,
    scratch_types=[
        pltpu.VMEM((_C,), jnp.int32),
        pltpu.VMEM((_C,), jnp.int32),
        pltpu.VMEM((_C, _D), jnp.float32),
        pltpu.VMEM((_C, _D), jnp.float32),
        pltpu.SemaphoreType.DMA,
        pltpu.SemaphoreType.DMA,
        pltpu.SemaphoreType.DMA,
        pltpu.SemaphoreType.DMA,
    ],
    compiler_params=pltpu.CompilerParams(use_tc_tiling_on_sc=False),
)


@jax.jit
def kernel(tokens, table):
    flat = tokens.reshape(-1).astype(jnp.int32)
    out = _gather(flat, table)
    return out.reshape(tokens.shape + (_D,))


# speedup vs baseline: 1.4739x; 1.0009x over previous
import math

import jax
import jax.numpy as jnp
from jax import lax
from jax.experimental import pallas as pl
from jax.experimental.pallas import tpu as pltpu
from jax.experimental.pallas import tpu_sc as plsc

_D = 32
_VOCAB = 1000000
_VOCABDIV4 = _VOCAB // 4
_B = 4096 * 200
_NC, _NS = 2, 16
_NW = _NC * _NS
_BPW = _B // _NW
_C = 1600
_NCH = _BPW // _C
_SCALE = math.sqrt(float(_D))


def _scale_buf(rows_v):
    @plsc.parallel_loop(0, _C, 1, unroll=8)
    def _s(i):
        rows_v[i, pl.ds(0, 16)] = rows_v[i, pl.ds(0, 16)] * _SCALE
        rows_v[i, pl.ds(16, 16)] = rows_v[i, pl.ds(16, 16)] * _SCALE


def _body(tok_hbm, table_hbm, out_hbm,
          idx0, idx1, rows0, rows1, gsem0, gsem1, osem0, osem1):
    idx = (idx0, idx1)
    rows = (rows0, rows1)
    gsem = (gsem0, gsem1)
    osem = (osem0, osem1)
    wid = lax.axis_index("s") * _NC + lax.axis_index("c")
    base = wid * _BPW

    gd = [None, None]
    od = [None, None]
    for c in range(_NCH):
        b = c & 1
        if od[b] is not None:
            od[b].wait()
        pltpu.sync_copy(tok_hbm.at[pl.ds(base + c * _C, _C)], idx[b])
        gd[b] = pltpu.async_copy(table_hbm.at[idx[b]], rows[b], gsem[b])
        if c > 0:
            pb = (c - 1) & 1
            gd[pb].wait()
            _scale_buf(rows[pb])
            od[pb] = pltpu.async_copy(
                rows[pb], out_hbm.at[pl.ds(base + (c - 1) * _C, _C)], osem[pb])
    pb = (_NCH - 1) & 1
    gd[pb].wait()
    _scale_buf(rows[pb])
    od[pb] = pltpu.async_copy(
        rows[pb], out_hbm.at[pl.ds(base + (_NCH - 1) * _C, _C)], osem[pb])
    od[0].wait()
    od[1].wait()


_mesh = plsc.VectorSubcoreMesh(
    core_axis_name="c", subcore_axis_name="s", num_cores=_NC, num_subcores=_NS
)

_gather = pl.kernel(
    _body,
    out_type=jax.ShapeDtypeStruct((_B, _D), jnp.float32),
    mesh=_mesh,
    scratch_types=[
        pltpu.VMEM((_C,), jnp.int32),
        pltpu.VMEM((_C,), jnp.int32),
        pltpu.VMEM((_C, _D), jnp.float32),
        pltpu.VMEM((_C, _D), jnp.float32),
        pltpu.SemaphoreType.DMA,
        pltpu.SemaphoreType.DMA,
        pltpu.SemaphoreType.DMA,
        pltpu.SemaphoreType.DMA,
    ],
    compiler_params=pltpu.CompilerParams(use_tc_tiling_on_sc=False),
)


@jax.jit
def kernel(tokens, table):
    flat = tokens.reshape(-1).astype(jnp.int32)
    t128 = jax.lax.optimization_barrier(table.reshape(_VOCABDIV4, 128))
    out = _gather(flat, t128.reshape(_VOCAB, _D))
    return out.reshape(tokens.shape + (_D,))


# speedup vs baseline: 1.5487x; 1.0508x over previous
import math

import jax
import jax.numpy as jnp
from jax import lax
from jax.experimental import pallas as pl
from jax.experimental.pallas import tpu as pltpu
from jax.experimental.pallas import tpu_sc as plsc

_D = 32
_VOCAB = 1000000
_SEQ = 200
_BATCH = 4096
_NC, _NS = 2, 16
_NW = _NC * _NS
_U = 256
_GPS = _BATCH // _U
_UNITS = _SEQ * _GPS
_UPW = _UNITS // _NW
_SCALE = math.sqrt(float(_D))


def _body(tok_hbm, t128_hbm, out_hbm,
          idx0, idx1, q0, q1, rows0, rows1, stg,
          gsem0, gsem1, osem):
    idx = (idx0, idx1)
    q = (q0, q1)
    rows = (rows0, rows1)
    gsem = (gsem0, gsem1)
    wid = lax.axis_index("s") * _NC + lax.axis_index("c")
    u0 = wid * _UPW

    def load_unit(u, b):
        g = (u0 + u) & (_GPS - 1)
        s = (u0 + u) >> 4
        off = s * _BATCH + g * _U
        pltpu.sync_copy(tok_hbm.at[pl.ds(off, _U)], idx[b])

        @plsc.parallel_loop(0, _U // 16, 1, unroll=4)
        def _q(i):
            q[b][pl.ds(i * 16, 16)] = idx[b][pl.ds(i * 16, 16)] >> 2

        pltpu.async_copy(t128_hbm.at[q[b]], rows[b], gsem[b])

    def wait_gather(b):
        pltpu.make_async_copy(t128_hbm.at[q[b]], rows[b], gsem[b]).wait()

    def transpose_unit(b):
        iota = lax.iota(jnp.int32, 16)

        @plsc.parallel_loop(0, 16, 1)
        def _grp(grp):
            c0 = (grp & 7) * 16
            cp = grp >> 3
            tok = idx[b][pl.ds(grp * 16, 16)]
            sub = (tok & 3) << 5
            rowv = grp * 16 + iota
            for e in range(_D):
                v = plsc.load_gather(rows[b], [rowv, sub + e])
                stg[e >> 3, cp, e & 7, pl.ds(c0, 16)] = v * _SCALE

    def flush_unit(u):
        g = (u0 + u) & (_GPS - 1)
        s = (u0 + u) >> 4
        for r4 in range(4):
            pltpu.async_copy(stg.at[r4], out_hbm.at[s, r4, pl.ds(g * 2, 2)], osem)

    def drain_flush(u):
        g = (u0 + u) & (_GPS - 1)
        s = (u0 + u) >> 4
        for r4 in range(4):
            pltpu.make_async_copy(
                stg.at[r4], out_hbm.at[s, r4, pl.ds(g * 2, 2)], osem).wait()

    load_unit(0, 0)

    @pl.loop(0, _UPW, step=2)
    def _units(i):
        for ph in range(2):
            u = i + ph
            b = ph

            @pl.when(u < _UPW - 1)
            def _prefetch():
                load_unit(u + 1, b ^ 1)

            wait_gather(b)

            @pl.when(u > 0)
            def _drain():
                drain_flush(u)

            transpose_unit(b)
            flush_unit(u)

    drain_flush(0)


_mesh = plsc.VectorSubcoreMesh(
    core_axis_name="c", subcore_axis_name="s", num_cores=_NC, num_subcores=_NS
)

_gather = pl.kernel(
    _body,
    out_type=jax.ShapeDtypeStruct((_SEQ, 4, _BATCH // 128, 8, 128), jnp.float32),
    mesh=_mesh,
    scratch_types=[
        pltpu.VMEM((_U,), jnp.int32),
        pltpu.VMEM((_U,), jnp.int32),
        pltpu.VMEM((_U,), jnp.int32),
        pltpu.VMEM((_U,), jnp.int32),
        pltpu.VMEM((_U, 128), jnp.float32),
        pltpu.VMEM((_U, 128), jnp.float32),
        pltpu.VMEM((4, 2, 8, 128), jnp.float32),
        pltpu.SemaphoreType.DMA,
        pltpu.SemaphoreType.DMA,
        pltpu.SemaphoreType.DMA,
    ],
    compiler_params=pltpu.CompilerParams(
        use_tc_tiling_on_sc=False, needs_layout_passes=False
    ),
)


@jax.jit
def kernel(tokens, table):
    tflat = tokens.astype(jnp.int32).T.reshape(-1)
    t128 = jax.lax.optimization_barrier(table.reshape(_VOCAB // 4, 128))
    out5 = _gather(tflat, t128)
    return out5.transpose(2, 4, 0, 1, 3).reshape(_BATCH, _SEQ, _D)


# speedup vs baseline: 3.2326x; 2.0872x over previous
import math

import jax
import jax.numpy as jnp
from jax import lax
from jax.experimental import pallas as pl
from jax.experimental.pallas import tpu as pltpu
from jax.experimental.pallas import tpu_sc as plsc

_D = 32
_VOCAB = 1000000
_SEQ = 200
_BATCH = 4096
_NC, _NS = 2, 16
_NW = _NC * _NS
_SCALE = math.sqrt(float(_D))

_mesh = plsc.VectorSubcoreMesh(
    core_axis_name="c", subcore_axis_name="s", num_cores=_NC, num_subcores=_NS
)

_KB = 7812
_BPW = _KB // _NW


def _lin_body(tt_hbm, tail_hbm, lin_hbm, sin0, sin1, sout0, sout1,
              isem0, isem1, osem0, osem1):
    sin = (sin0, sin1)
    sout = (sout0, sout1)
    isem = (isem0, isem1)
    osem = (osem0, osem1)
    wid = lax.axis_index("s") * _NC + lax.axis_index("c")
    iota = lax.iota(jnp.int32, 16)

    def start_in(k, b, width):
        c0 = (wid * _BPW + k) * 128 if width == 128 else _KB * 128
        for r4 in range(4):
            pltpu.async_copy(
                tt_hbm.at[pl.ds(8 * r4, 8), pl.ds(c0, width)],
                sin[b].at[r4, slice(None), pl.ds(0, width)], isem[b])

    def drain_in(b, width):
        for r4 in range(4):
            pltpu.make_async_copy(
                tt_hbm.at[pl.ds(0, 8), pl.ds(0, width)],
                sin[b].at[0, slice(None), pl.ds(0, width)], isem[b]).wait()

    def transpose(b, nv):
        @plsc.parallel_loop(0, nv // 16, 1)
        def _vg(v0g):
            vv = v0g * 16 + iota
            base = vv << 5
            for e0 in range(_D):
                ev = (e0 + iota) & 31
                val = plsc.load_gather(sin[b], [ev >> 3, ev & 7, vv])
                plsc.store_scatter(sout[b], [base + ev], val * _SCALE)

    def start_out(k, b, nv):
        c0 = (wid * _BPW + k) * 128 if nv == 128 else _KB * 128
        pltpu.async_copy(
            sout[b].at[pl.ds(0, nv * _D)],
            lin_hbm.at[pl.ds(c0 * _D, nv * _D)], osem[b])

    def drain_out(b):
        pltpu.make_async_copy(
            sout[b].at[pl.ds(0, 128 * _D)],
            lin_hbm.at[pl.ds(0, 128 * _D)], osem[b]).wait()

    start_in(0, 0, 128)

    @pl.loop(0, _BPW, step=2)
    def _blocks(k):
        for ph in range(2):
            u = k + ph
            b = ph

            @pl.when(u < _BPW - 1)
            def _pf():
                start_in(u + 1, b ^ 1, 128)

            drain_in(b, 128)

            @pl.when(u >= 2)
            def _dr():
                drain_out(b)

            transpose(b, 128)
            start_out(u, b, 128)

    drain_out(0)
    drain_out(1)

    @pl.when(wid >= _NW - 4)
    def _extra():
        cx = (_KB - 4 + wid - (_NW - 4)) * 128
        for r4 in range(4):
            pltpu.async_copy(
                tt_hbm.at[pl.ds(8 * r4, 8), pl.ds(cx, 128)], sin[0].at[r4],
                isem[0])
        drain_in(0, 128)
        transpose(0, 128)
        pltpu.async_copy(
            sout[0].at[pl.ds(0, 128 * _D)],
            lin_hbm.at[pl.ds(cx * _D, 128 * _D)], osem[0])
        drain_out(0)

    @pl.when(wid == _NW - 5)
    def _tail():
        pltpu.sync_copy(tail_hbm, sout[0].at[pl.ds(0, 64 * _D)])
        pltpu.sync_copy(sout[0].at[pl.ds(0, 64 * _D)],
                        lin_hbm.at[pl.ds(_KB * 128 * _D, 64 * _D)])


_linearize = pl.kernel(
    _lin_body,
    out_type=jax.ShapeDtypeStruct((_VOCAB * _D,), jnp.float32),
    mesh=_mesh,
    scratch_types=[
        pltpu.VMEM((4, 8, 128), jnp.float32),
        pltpu.VMEM((4, 8, 128), jnp.float32),
        pltpu.VMEM((128 * _D,), jnp.float32),
        pltpu.VMEM((128 * _D,), jnp.float32),
        pltpu.SemaphoreType.DMA,
        pltpu.SemaphoreType.DMA,
        pltpu.SemaphoreType.DMA,
        pltpu.SemaphoreType.DMA,
    ],
    compiler_params=pltpu.CompilerParams(
        use_tc_tiling_on_sc=True, needs_layout_passes=False
    ),
)

_U = 512
_GPS = _BATCH // _U
_UPW = _SEQ * _GPS // _NW


def _gat_body(tok_hbm, t32_hbm, out_hbm,
              idx0, idx1, rows0, rows1, stg, gsem0, gsem1, osem):
    idx = (idx0, idx1)
    rows = (rows0, rows1)
    gsem = (gsem0, gsem1)
    wid = lax.axis_index("s") * _NC + lax.axis_index("c")
    u0 = wid * _UPW
    iota = lax.iota(jnp.int32, 16)

    def load_unit(u, b):
        g = (u0 + u) & (_GPS - 1)
        s = (u0 + u) >> 3
        pltpu.sync_copy(tok_hbm.at[pl.ds(s * _BATCH + g * _U, _U)], idx[b])
        pltpu.async_copy(t32_hbm.at[idx[b]], rows[b], gsem[b])

    def wait_gather(b):
        pltpu.make_async_copy(t32_hbm.at[idx[b]], rows[b], gsem[b]).wait()

    def transpose_unit(b):
        @plsc.parallel_loop(0, _U // 16, 1)
        def _grp(grp):
            cv = ((grp & 7) * 16) + iota
            cpv = lax.broadcast(grp >> 3, (16,))
            rowv = grp * 16 + iota
            for e0 in range(_D):
                ev = (e0 + iota) & 31
                v = plsc.load_gather(rows[b], [rowv, ev])
                plsc.store_scatter(stg, [ev >> 3, cpv, ev & 7, cv], v)

    def flush_unit(u):
        g = (u0 + u) & (_GPS - 1)
        s = (u0 + u) >> 3
        for r4 in range(4):
            pltpu.async_copy(stg.at[r4], out_hbm.at[s, r4, pl.ds(g * 4, 4)],
                             osem)

    def drain_flush(u):
        g = (u0 + u) & (_GPS - 1)
        s = (u0 + u) >> 3
        for r4 in range(4):
            pltpu.make_async_copy(
                stg.at[r4], out_hbm.at[s, r4, pl.ds(g * 4, 4)], osem).wait()

    load_unit(0, 0)

    @pl.loop(0, _UPW, step=2)
    def _units(i):
        for ph in range(2):
            u = i + ph
            b = ph

            @pl.when(u < _UPW - 1)
            def _prefetch():
                load_unit(u + 1, b ^ 1)

            wait_gather(b)

            @pl.when(u > 0)
            def _drain():
                drain_flush(u)

            transpose_unit(b)
            flush_unit(u)

    drain_flush(0)


_gather = pl.kernel(
    _gat_body,
    out_type=jax.ShapeDtypeStruct((_SEQ, 4, _BATCH // 128, 8, 128), jnp.float32),
    mesh=_mesh,
    scratch_types=[
        pltpu.VMEM((_U,), jnp.int32),
        pltpu.VMEM((_U,), jnp.int32),
        pltpu.VMEM((_U, _D), jnp.float32),
        pltpu.VMEM((_U, _D), jnp.float32),
        pltpu.VMEM((4, 4, 8, 128), jnp.float32),
        pltpu.SemaphoreType.DMA,
        pltpu.SemaphoreType.DMA,
        pltpu.SemaphoreType.DMA,
    ],
    compiler_params=pltpu.CompilerParams(
        use_tc_tiling_on_sc=False, needs_layout_passes=False
    ),
)


@jax.jit
def kernel(tokens, table):
    tflat = tokens.astype(jnp.int32).T.reshape(-1)
    t_t = table.T
    t_tail = jax.lax.optimization_barrier(
        (table[_KB * 128:, :] * _SCALE).reshape(64 * _D))
    t32 = _linearize(t_t, t_tail).reshape(_VOCAB, _D)
    out5 = _gather(tflat, t32)
    return out5.transpose(2, 4, 0, 1, 3).reshape(_BATCH, _SEQ, _D)


# speedup vs baseline: 4.3226x; 1.3372x over previous
import math

import jax
import jax.numpy as jnp
from jax import lax
from jax.experimental import pallas as pl
from jax.experimental.pallas import tpu as pltpu
from jax.experimental.pallas import tpu_sc as plsc

_D = 32
_VOCAB = 1000000
_SEQ = 200
_BATCH = 4096
_NC, _NS = 2, 16
_NW = _NC * _NS
_SCALE = math.sqrt(float(_D))

_mesh = plsc.VectorSubcoreMesh(
    core_axis_name="c", subcore_axis_name="s", num_cores=_NC, num_subcores=_NS
)

_KB = 3906
_BPW = _KB // _NW


def _lin_body(tt_hbm, tail_hbm, lin_hbm, sin0, sin1, sout0, sout1,
              isem0, isem1, osem0, osem1):
    sin = (sin0, sin1)
    sout = (sout0, sout1)
    isem = (isem0, isem1)
    osem = (osem0, osem1)
    wid = lax.axis_index("s") * _NC + lax.axis_index("c")
    iota = lax.iota(jnp.int32, 16)

    def start_in(c0, b):
        for r4 in range(4):
            pltpu.async_copy(
                tt_hbm.at[pl.ds(8 * r4, 8), pl.ds(c0, 256)], sin[b].at[r4],
                isem[b])

    def drain_in(b):
        for r4 in range(4):
            pltpu.make_async_copy(
                tt_hbm.at[pl.ds(0, 8), pl.ds(0, 256)], sin[b].at[0],
                isem[b]).wait()

    def transpose(b):
        @plsc.parallel_loop(0, 16, 1)
        def _vg(v0g):
            vv = v0g * 16 + iota
            base = vv << 5
            for e0 in range(_D):
                ev = (e0 + iota) & 31
                val = plsc.load_gather(sin[b], [ev >> 3, ev & 7, vv])
                plsc.store_scatter(sout[b], [base + ev], val * _SCALE)

    def start_out(c0, b):
        pltpu.async_copy(
            sout[b], lin_hbm.at[pl.ds(c0 * _D, 256 * _D)], osem[b])

    def drain_out(b):
        pltpu.make_async_copy(
            sout[b], lin_hbm.at[pl.ds(0, 256 * _D)], osem[b]).wait()

    start_in(wid * _BPW * 256, 0)

    @pl.loop(0, _BPW, step=2)
    def _blocks(k):
        for ph in range(2):
            u = k + ph
            b = ph

            @pl.when(u < _BPW - 1)
            def _pf():
                start_in((wid * _BPW + u + 1) * 256, b ^ 1)

            drain_in(b)

            @pl.when(u >= 2)
            def _dr():
                drain_out(b)

            transpose(b)
            start_out((wid * _BPW + u) * 256, b)

    drain_out(0)
    drain_out(1)

    @pl.when(wid >= _NW - 2)
    def _extra():
        cx = (_KB - 2 + wid - (_NW - 2)) * 256
        start_in(cx, 0)
        drain_in(0)
        transpose(0)
        start_out(cx, 0)
        drain_out(0)

    @pl.when(wid == _NW - 5)
    def _tail():
        pltpu.sync_copy(tail_hbm, sout[0].at[pl.ds(0, 64 * _D)])
        pltpu.sync_copy(sout[0].at[pl.ds(0, 64 * _D)],
                        lin_hbm.at[pl.ds(_KB * 256 * _D, 64 * _D)])


_linearize = pl.kernel(
    _lin_body,
    out_type=jax.ShapeDtypeStruct((_VOCAB * _D,), jnp.float32),
    mesh=_mesh,
    scratch_types=[
        pltpu.VMEM((4, 8, 256), jnp.float32),
        pltpu.VMEM((4, 8, 256), jnp.float32),
        pltpu.VMEM((256 * _D,), jnp.float32),
        pltpu.VMEM((256 * _D,), jnp.float32),
        pltpu.SemaphoreType.DMA,
        pltpu.SemaphoreType.DMA,
        pltpu.SemaphoreType.DMA,
        pltpu.SemaphoreType.DMA,
    ],
    compiler_params=pltpu.CompilerParams(
        use_tc_tiling_on_sc=True, needs_layout_passes=False
    ),
)

_U = 512
_GPS = _BATCH // _U
_UPW = _SEQ * _GPS // _NW


def _gat_body(tok_hbm, t32_hbm, out_hbm,
              idx0, idx1, rows0, rows1, stg, gsem0, gsem1, osem):
    idx = (idx0, idx1)
    rows = (rows0, rows1)
    gsem = (gsem0, gsem1)
    wid = lax.axis_index("s") * _NC + lax.axis_index("c")
    u0 = wid * _UPW
    iota = lax.iota(jnp.int32, 16)

    def load_unit(u, b):
        g = (u0 + u) & (_GPS - 1)
        s = (u0 + u) >> 3
        pltpu.sync_copy(tok_hbm.at[pl.ds(s * _BATCH + g * _U, _U)], idx[b])
        pltpu.async_copy(t32_hbm.at[idx[b]], rows[b], gsem[b])

    def wait_gather(b):
        pltpu.make_async_copy(t32_hbm.at[idx[b]], rows[b], gsem[b]).wait()

    def transpose_unit(b):
        @plsc.parallel_loop(0, _U // 16, 1)
        def _grp(grp):
            cv = ((grp & 7) * 16) + iota
            cpv = lax.broadcast(grp >> 3, (16,))
            rowv = grp * 16 + iota
            for e0 in range(_D):
                ev = (e0 + iota) & 31
                v = plsc.load_gather(rows[b], [rowv, ev])
                plsc.store_scatter(stg, [ev >> 3, cpv, ev & 7, cv], v)

    def flush_unit(u):
        g = (u0 + u) & (_GPS - 1)
        s = (u0 + u) >> 3
        for r4 in range(4):
            pltpu.async_copy(stg.at[r4], out_hbm.at[s, r4, pl.ds(g * 4, 4)],
                             osem)

    def drain_flush(u):
        g = (u0 + u) & (_GPS - 1)
        s = (u0 + u) >> 3
        for r4 in range(4):
            pltpu.make_async_copy(
                stg.at[r4], out_hbm.at[s, r4, pl.ds(g * 4, 4)], osem).wait()

    load_unit(0, 0)

    @pl.loop(0, _UPW, step=2)
    def _units(i):
        for ph in range(2):
            u = i + ph
            b = ph

            @pl.when(u < _UPW - 1)
            def _prefetch():
                load_unit(u + 1, b ^ 1)

            wait_gather(b)

            @pl.when(u > 0)
            def _drain():
                drain_flush(u)

            transpose_unit(b)
            flush_unit(u)

    drain_flush(0)


_gather = pl.kernel(
    _gat_body,
    out_type=jax.ShapeDtypeStruct((_SEQ, 4, _BATCH // 128, 8, 128), jnp.float32),
    mesh=_mesh,
    scratch_types=[
        pltpu.VMEM((_U,), jnp.int32),
        pltpu.VMEM((_U,), jnp.int32),
        pltpu.VMEM((_U, _D), jnp.float32),
        pltpu.VMEM((_U, _D), jnp.float32),
        pltpu.VMEM((4, 4, 8, 128), jnp.float32),
        pltpu.SemaphoreType.DMA,
        pltpu.SemaphoreType.DMA,
        pltpu.SemaphoreType.DMA,
    ],
    compiler_params=pltpu.CompilerParams(
        use_tc_tiling_on_sc=False, needs_layout_passes=False
    ),
)


@jax.jit
def kernel(tokens, table):
    tflat = tokens.astype(jnp.int32).T.reshape(-1)
    t_t = table.T
    t_tail = jax.lax.optimization_barrier(
        (table[_KB * 256:, :] * _SCALE).reshape(64 * _D))
    t32 = _linearize(t_t, t_tail).reshape(_VOCAB, _D)
    out5 = _gather(tflat, t32)
    return out5.transpose(2, 4, 0, 1, 3).reshape(_BATCH, _SEQ, _D)
